# baseline (speedup 1.0000x reference)
import jax
import jax.numpy as jnp
from jax import lax
from jax.experimental import pallas as pl
from jax.experimental.pallas import tpu as pltpu
from jax.experimental.pallas import tpu_sc as plsc

D = 64
LN_EPS = 1e-5
NC = 2
NS = 16
NW = NC * NS
TPAD = 273
V = 1000000
VBLK = V // 128
VBLK_MAIN = 7808
BPW = VBLK_MAIN // NW

_GATHER_DNUMS = lax.GatherDimensionNumbers(
    offset_dims=(), collapsed_slice_dims=(0,), start_index_map=(0,))


def _lane_sum(x):
    lane = lax.iota(jnp.int32, 16)
    for s in (1, 2, 4, 8):
        p = (lane ^ s).reshape(16, 1)
        x = x + lax.gather(x, p, _GATHER_DNUMS, (1,),
                           mode=lax.GatherScatterMode.PROMISE_IN_BOUNDS)
    return x


def _ln_row_t(gbuf, tbuf, r, g_regs, b_regs, kbase):
    r2 = 2 * r
    v = [gbuf[r2, pl.ds(0, 16)], gbuf[r2, pl.ds(16, 16)],
         gbuf[r2 + 1, pl.ds(0, 16)], gbuf[r2 + 1, pl.ds(16, 16)]]
    s = _lane_sum(v[0] + v[1] + v[2] + v[3])
    ss = _lane_sum(v[0] * v[0] + (v[1] * v[1] + (v[2] * v[2] + v[3] * v[3])))
    mean = s * (1.0 / 64.0)
    var = ss * (1.0 / 64.0) - mean * mean
    x = var + LN_EPS
    i = lax.bitcast_convert_type(x, jnp.int32)
    i = jnp.int32(0x5F3759DF) - lax.shift_right_logical(i, 1)
    y = lax.bitcast_convert_type(i, jnp.float32)
    xh = 0.5 * x
    y = y * (1.5 - xh * y * y)
    y = y * (1.5 - xh * y * y)
    nb = -mean * y
    rv = jnp.full((16,), r, jnp.int32)
    for k in range(4):
        o = (v[k] * y + nb) * g_regs[k] + b_regs[k]
        plsc.store_scatter(tbuf, [kbase[k], rv], o)


def _make_transpose_call():
    mesh = plsc.VectorSubcoreMesh(core_axis_name="c", subcore_axis_name="s")

    def body(tv_hbm, tt_hbm, v0, v1, t0, t1, u0, u1, sr0, sr1, sw0, sw1):
        wid = lax.axis_index("s") * NC + lax.axis_index("c")
        base_ib = wid * BPW
        lane = lax.iota(jnp.int32, 16)
        qv = [(16 * m + lane) >> 1 for m in range(8)]
        cb = [((16 * m + lane) & 1) * 64 for m in range(8)]
        bufs = ((v0, t0, u0, sr0, sw0), (v1, t1, u1, sr1, sw1))

        def fire_read(ib, vb, sr):
            pltpu.async_copy(tv_hbm.at[:, pl.ds(ib * 128, 128)], vb, sr)

        def wait_read(ib, vb, sr):
            pltpu.make_async_copy(
                tv_hbm.at[:, pl.ds(ib * 128, 128)], vb, sr).wait()

        def fire_write(ib, ub, sw):
            pltpu.async_copy(ub, tt_hbm.at[pl.ds(ib * 64, 64)], sw)

        def wait_write(ib, ub, sw):
            pltpu.make_async_copy(
                ub, tt_hbm.at[pl.ds(ib * 64, 64)], sw).wait()

        def transpose(vb, tb, ub, nd):
            @pl.loop(0, nd, unroll=2)
            def _d(d):
                for m in range(8):
                    x = vb[d, pl.ds(16 * m, 16)]
                    plsc.store_scatter(tb, [qv[m], cb[m] + d], x)

            @pl.loop(0, nd, unroll=2)
            def _q(q):
                for m in range(8):
                    ub[q, pl.ds(16 * m, 16)] = tb[q, pl.ds(16 * m, 16)]

        def step(c, p, first):
            v_m, t_m, u_m, sr_m, sw_m = bufs[p]
            v_o, t_o, u_o, sr_o, sw_o = bufs[1 - p]
            ib = base_ib + c
            if not first:
                wait_write(ib - 1, u_o, sw_o)

            @pl.when(c + 1 < BPW)
            def _():
                fire_read(ib + 1, v_o, sr_o)

            wait_read(ib, v_m, sr_m)
            transpose(v_m, t_m, u_m, D)
            fire_write(ib, u_m, sw_m)

        fire_read(base_ib, v0, sr0)
        step(0, 0, first=True)

        @pl.loop(1, BPW)
        def _blk(c):
            @pl.when((c & 1) == 1)
            def _():
                step(c, 1, first=False)

            @pl.when((c & 1) == 0)
            def _():
                step(c, 0, first=False)

        last = (BPW - 1) & 1
        wait_write(base_ib + BPW - 1, bufs[last][2], bufs[last][4])

        @pl.when(wid < VBLK - VBLK_MAIN)
        def _():
            ib = VBLK_MAIN + wid
            pltpu.async_copy(tv_hbm.at[:, pl.ds(ib * 128, 128)], v0, sr0)
            pltpu.make_async_copy(
                tv_hbm.at[:, pl.ds(ib * 128, 128)], v0, sr0).wait()
            transpose(v0, t0, u0, D)
            pltpu.sync_copy(u0, tt_hbm.at[pl.ds(ib * 64, 64)])

    return pl.kernel(
        body,
        out_type=jax.ShapeDtypeStruct((V // 2, 128), jnp.float32),
        mesh=mesh,
        scratch_types=[
            pltpu.VMEM((D, 128), jnp.float32),
            pltpu.VMEM((D, 128), jnp.float32),
            pltpu.VMEM((D, 129), jnp.float32),
            pltpu.VMEM((D, 129), jnp.float32),
            pltpu.VMEM((D, 128), jnp.float32),
            pltpu.VMEM((D, 128), jnp.float32),
            pltpu.SemaphoreType.DMA,
            pltpu.SemaphoreType.DMA,
            pltpu.SemaphoreType.DMA,
            pltpu.SemaphoreType.DMA,
        ],
        compiler_params=pltpu.CompilerParams(
            use_tc_tiling_on_sc=True, needs_layout_passes=False),
    )


def _make_gather_call(batch, n_l, chunk):
    stripe = batch // NW
    halves = stripe // chunk
    n_chunks = n_l * halves
    mesh = plsc.VectorSubcoreMesh(core_axis_name="c", subcore_axis_name="s")

    def body(x_hbm, tt_hbm, g_hbm, b_hbm, out_hbm,
             idx_all, idx2, g0, g1, t0, t1, g_v, b_v,
             sg0, sg1, so0, so1):
        wid = lax.axis_index("s") * NC + lax.axis_index("c")
        pltpu.sync_copy(x_hbm.at[wid], idx_all)
        pltpu.sync_copy(g_hbm, g_v)
        pltpu.sync_copy(b_hbm, b_v)
        g_regs = [g_v[pl.ds(16 * k, 16)] for k in range(4)]
        b_regs = [b_v[pl.ds(16 * k, 16)] for k in range(4)]
        lane = lax.iota(jnp.int32, 16)
        kbase = [lane + (16 * k) for k in range(4)]
        pos = [lane * 2 + 32 * m for m in range(chunk // 16)]
        bufs = ((g0, t0, sg0, so0), (g1, t1, sg1, so1))

        def stage_idx(c, p):
            pv = jnp.full((16,), p, jnp.int32)
            for m in range(chunk // 16):
                iv = idx_all[c >> 1, c & (halves - 1), pl.ds(16 * m, 16)]
                e = iv * 2
                plsc.store_scatter(idx2, [pv, pos[m]], e)
                plsc.store_scatter(idx2, [pv, pos[m] + 1], e + 1)

        def fire_gather(p, g, sg):
            pltpu.async_copy(tt_hbm.at[idx2.at[p]], g, sg)

        def wait_gather(p, g, sg):
            pltpu.make_async_copy(tt_hbm.at[idx2.at[p]], g, sg).wait()

        def out_slice(c):
            col = wid * halves + (c & (halves - 1))
            return out_hbm.at[c >> 1, :, pl.ds(col * chunk, chunk)]

        def fire_out(c, t, so):
            pltpu.async_copy(t.at[:, pl.ds(0, chunk)], out_slice(c), so)

        def wait_out(c, t, so):
            pltpu.make_async_copy(
                t.at[:, pl.ds(0, chunk)], out_slice(c), so).wait()

        def compute(g, t):
            @plsc.parallel_loop(0, chunk, unroll=4)
            def _row(r):
                _ln_row_t(g, t, r, g_regs, b_regs, kbase)

        def step(c, p, first):
            g_m, t_m, sg_m, so_m = bufs[p]
            g_o, t_o, sg_o, so_o = bufs[1 - p]
            if not first:
                wait_out(c - 1, t_o, so_o)

            @pl.when(c + 1 < n_chunks)
            def _():
                stage_idx(c + 1, 1 - p)
                fire_gather(1 - p, g_o, sg_o)

            wait_gather(p, g_m, sg_m)
            compute(g_m, t_m)
            fire_out(c, t_m, so_m)

        stage_idx(0, 0)
        fire_gather(0, g0, sg0)
        step(0, 0, first=True)

        @pl.loop(1, n_chunks)
        def _chunk(c):
            @pl.when((c & 1) == 1)
            def _():
                step(c, 1, first=False)

            @pl.when((c & 1) == 0)
            def _():
                step(c, 0, first=False)

        last = (n_chunks - 1) & 1
        wait_out(n_chunks - 1, bufs[last][1], bufs[last][3])

    return pl.kernel(
        body,
        out_type=jax.ShapeDtypeStruct((n_l, D, batch), jnp.float32),
        mesh=mesh,
        scratch_types=[
            pltpu.VMEM((n_l, halves, chunk), jnp.int32),
            pltpu.VMEM((2, 2 * chunk), jnp.int32),
            pltpu.VMEM((2 * chunk, 32), jnp.float32),
            pltpu.VMEM((2 * chunk, 32), jnp.float32),
            pltpu.VMEM((D, TPAD), jnp.float32),
            pltpu.VMEM((D, TPAD), jnp.float32),
            pltpu.VMEM((D,), jnp.float32),
            pltpu.VMEM((D,), jnp.float32),
            pltpu.SemaphoreType.DMA,
            pltpu.SemaphoreType.DMA,
            pltpu.SemaphoreType.DMA,
            pltpu.SemaphoreType.DMA,
        ],
        compiler_params=pltpu.CompilerParams(
            use_tc_tiling_on_sc=False, needs_layout_passes=False),
    )


def kernel(x, table, gamma, beta):
    b, l = x.shape
    chunk = 256
    tt = _make_transpose_call()(table.T)
    tail = table[VBLK * 128:].reshape(-1, 128)
    tt = lax.dynamic_update_slice(tt, tail, (VBLK * D, 0))
    tt2 = tt.reshape(2 * V, 32)
    xt = x.T.reshape(l, NW, b // (NW * chunk), chunk).transpose(1, 0, 2, 3)
    out = _make_gather_call(b, l, chunk)(xt, tt2, gamma, beta)
    return out.transpose(2, 0, 1)


# speedup vs baseline: 1.3993x; 1.3993x over previous
import jax
import jax.numpy as jnp
from jax import lax
from jax.experimental import pallas as pl
from jax.experimental.pallas import tpu as pltpu
from jax.experimental.pallas import tpu_sc as plsc

D = 64
LN_EPS = 1e-5
NC = 2
NS = 16
NW = NC * NS
TPAD = 273
V = 1000000
VBLK = V // 128
VBLK_MAIN = 7808
BPW = VBLK_MAIN // NW

_GATHER_DNUMS = lax.GatherDimensionNumbers(
    offset_dims=(), collapsed_slice_dims=(0,), start_index_map=(0,))


def _lane_sum(x):
    lane = lax.iota(jnp.int32, 16)
    for s in (1, 2, 4, 8):
        p = (lane ^ s).reshape(16, 1)
        x = x + lax.gather(x, p, _GATHER_DNUMS, (1,),
                           mode=lax.GatherScatterMode.PROMISE_IN_BOUNDS)
    return x


def _ln_row_t(gbuf, tbuf, r, g_regs, b_regs, kbase):
    r2 = 2 * r
    v = [gbuf[r2, pl.ds(0, 16)], gbuf[r2, pl.ds(16, 16)],
         gbuf[r2 + 1, pl.ds(0, 16)], gbuf[r2 + 1, pl.ds(16, 16)]]
    s = _lane_sum(v[0] + v[1] + v[2] + v[3])
    ss = _lane_sum(v[0] * v[0] + (v[1] * v[1] + (v[2] * v[2] + v[3] * v[3])))
    mean = s * (1.0 / 64.0)
    var = ss * (1.0 / 64.0) - mean * mean
    x = var + LN_EPS
    i = lax.bitcast_convert_type(x, jnp.int32)
    i = jnp.int32(0x5F3759DF) - lax.shift_right_logical(i, 1)
    y = lax.bitcast_convert_type(i, jnp.float32)
    xh = 0.5 * x
    y = y * (1.5 - xh * y * y)
    y = y * (1.5 - xh * y * y)
    nb = -mean * y
    rv = jnp.full((16,), r, jnp.int32)
    for k in range(4):
        o = (v[k] * y + nb) * g_regs[k] + b_regs[k]
        plsc.store_scatter(tbuf, [kbase[k], rv], o)


def _make_transpose_call():
    mesh = plsc.VectorSubcoreMesh(core_axis_name="c", subcore_axis_name="s")

    def body(tv_hbm, tt_hbm, v0, v1, t0, t1, u0, u1, sr0, sr1, sw0, sw1):
        wid = lax.axis_index("s") * NC + lax.axis_index("c")
        base_ib = wid * BPW
        lane = lax.iota(jnp.int32, 16)
        qv = [(16 * m + lane) >> 1 for m in range(8)]
        cb = [((16 * m + lane) & 1) * 64 for m in range(8)]
        bufs = ((v0, t0, u0, sr0, sw0), (v1, t1, u1, sr1, sw1))

        def fire_read(ib, vb, sr):
            pltpu.async_copy(tv_hbm.at[:, pl.ds(ib * 128, 128)], vb, sr)

        def wait_read(ib, vb, sr):
            pltpu.make_async_copy(
                tv_hbm.at[:, pl.ds(ib * 128, 128)], vb, sr).wait()

        def fire_write(ib, ub, sw):
            pltpu.async_copy(ub, tt_hbm.at[pl.ds(ib * 64, 64)], sw)

        def wait_write(ib, ub, sw):
            pltpu.make_async_copy(
                ub, tt_hbm.at[pl.ds(ib * 64, 64)], sw).wait()

        def transpose(vb, tb, ub, nd):
            @plsc.parallel_loop(0, nd, unroll=4)
            def _d(d):
                for m in range(8):
                    x = vb[d, pl.ds(16 * m, 16)]
                    plsc.store_scatter(tb, [qv[m], cb[m] + d], x)

            @plsc.parallel_loop(0, nd, unroll=4)
            def _q(q):
                for m in range(8):
                    ub[q, pl.ds(16 * m, 16)] = tb[q, pl.ds(16 * m, 16)]

        def step(c, p, first):
            v_m, t_m, u_m, sr_m, sw_m = bufs[p]
            v_o, t_o, u_o, sr_o, sw_o = bufs[1 - p]
            ib = base_ib + c
            if not first:
                wait_write(ib - 1, u_o, sw_o)

            @pl.when(c + 1 < BPW)
            def _():
                fire_read(ib + 1, v_o, sr_o)

            wait_read(ib, v_m, sr_m)
            transpose(v_m, t_m, u_m, D)
            fire_write(ib, u_m, sw_m)

        fire_read(base_ib, v0, sr0)
        step(0, 0, first=True)

        @pl.loop(1, BPW)
        def _blk(c):
            @pl.when((c & 1) == 1)
            def _():
                step(c, 1, first=False)

            @pl.when((c & 1) == 0)
            def _():
                step(c, 0, first=False)

        last = (BPW - 1) & 1
        wait_write(base_ib + BPW - 1, bufs[last][2], bufs[last][4])

        @pl.when(wid < VBLK - VBLK_MAIN)
        def _():
            ib = VBLK_MAIN + wid
            pltpu.async_copy(tv_hbm.at[:, pl.ds(ib * 128, 128)], v0, sr0)
            pltpu.make_async_copy(
                tv_hbm.at[:, pl.ds(ib * 128, 128)], v0, sr0).wait()
            transpose(v0, t0, u0, D)
            pltpu.sync_copy(u0, tt_hbm.at[pl.ds(ib * 64, 64)])

    return pl.kernel(
        body,
        out_type=jax.ShapeDtypeStruct((V // 2, 128), jnp.float32),
        mesh=mesh,
        scratch_types=[
            pltpu.VMEM((D, 128), jnp.float32),
            pltpu.VMEM((D, 128), jnp.float32),
            pltpu.VMEM((D, 129), jnp.float32),
            pltpu.VMEM((D, 129), jnp.float32),
            pltpu.VMEM((D, 128), jnp.float32),
            pltpu.VMEM((D, 128), jnp.float32),
            pltpu.SemaphoreType.DMA,
            pltpu.SemaphoreType.DMA,
            pltpu.SemaphoreType.DMA,
            pltpu.SemaphoreType.DMA,
        ],
        compiler_params=pltpu.CompilerParams(
            use_tc_tiling_on_sc=True, needs_layout_passes=False),
    )


def _make_gather_call(batch, n_l, chunk):
    stripe = batch // NW
    halves = stripe // chunk
    n_chunks = n_l * halves
    mesh = plsc.VectorSubcoreMesh(core_axis_name="c", subcore_axis_name="s")

    def body(x_hbm, tt_hbm, g_hbm, b_hbm, out_hbm,
             idx_all, idx2, g0, g1, t0, t1, g_v, b_v,
             sg0, sg1, so0, so1):
        wid = lax.axis_index("s") * NC + lax.axis_index("c")
        pltpu.sync_copy(x_hbm.at[wid], idx_all)
        pltpu.sync_copy(g_hbm, g_v)
        pltpu.sync_copy(b_hbm, b_v)
        g_regs = [g_v[pl.ds(16 * k, 16)] for k in range(4)]
        b_regs = [b_v[pl.ds(16 * k, 16)] for k in range(4)]
        lane = lax.iota(jnp.int32, 16)
        kbase = [lane + (16 * k) for k in range(4)]
        pos = [lane * 2 + 32 * m for m in range(chunk // 16)]
        bufs = ((g0, t0, sg0, so0), (g1, t1, sg1, so1))

        def stage_idx(c, p):
            pv = jnp.full((16,), p, jnp.int32)
            for m in range(chunk // 16):
                iv = idx_all[c >> 1, c & (halves - 1), pl.ds(16 * m, 16)]
                e = iv * 2
                plsc.store_scatter(idx2, [pv, pos[m]], e)
                plsc.store_scatter(idx2, [pv, pos[m] + 1], e + 1)

        def fire_gather(p, g, sg):
            pltpu.async_copy(tt_hbm.at[idx2.at[p]], g, sg)

        def wait_gather(p, g, sg):
            pltpu.make_async_copy(tt_hbm.at[idx2.at[p]], g, sg).wait()

        def out_slice(c):
            col = wid * halves + (c & (halves - 1))
            return out_hbm.at[c >> 1, :, pl.ds(col * chunk, chunk)]

        def fire_out(c, t, so):
            pltpu.async_copy(t.at[:, pl.ds(0, chunk)], out_slice(c), so)

        def wait_out(c, t, so):
            pltpu.make_async_copy(
                t.at[:, pl.ds(0, chunk)], out_slice(c), so).wait()

        def compute(g, t):
            @plsc.parallel_loop(0, chunk, unroll=4)
            def _row(r):
                _ln_row_t(g, t, r, g_regs, b_regs, kbase)

        def step(c, p, first):
            g_m, t_m, sg_m, so_m = bufs[p]
            g_o, t_o, sg_o, so_o = bufs[1 - p]
            if not first:
                wait_out(c - 1, t_o, so_o)

            @pl.when(c + 1 < n_chunks)
            def _():
                stage_idx(c + 1, 1 - p)
                fire_gather(1 - p, g_o, sg_o)

            wait_gather(p, g_m, sg_m)
            compute(g_m, t_m)
            fire_out(c, t_m, so_m)

        stage_idx(0, 0)
        fire_gather(0, g0, sg0)
        step(0, 0, first=True)

        @pl.loop(1, n_chunks)
        def _chunk(c):
            @pl.when((c & 1) == 1)
            def _():
                step(c, 1, first=False)

            @pl.when((c & 1) == 0)
            def _():
                step(c, 0, first=False)

        last = (n_chunks - 1) & 1
        wait_out(n_chunks - 1, bufs[last][1], bufs[last][3])

    return pl.kernel(
        body,
        out_type=jax.ShapeDtypeStruct((n_l, D, batch), jnp.float32),
        mesh=mesh,
        scratch_types=[
            pltpu.VMEM((n_l, halves, chunk), jnp.int32),
            pltpu.VMEM((2, 2 * chunk), jnp.int32),
            pltpu.VMEM((2 * chunk, 32), jnp.float32),
            pltpu.VMEM((2 * chunk, 32), jnp.float32),
            pltpu.VMEM((D, TPAD), jnp.float32),
            pltpu.VMEM((D, TPAD), jnp.float32),
            pltpu.VMEM((D,), jnp.float32),
            pltpu.VMEM((D,), jnp.float32),
            pltpu.SemaphoreType.DMA,
            pltpu.SemaphoreType.DMA,
            pltpu.SemaphoreType.DMA,
            pltpu.SemaphoreType.DMA,
        ],
        compiler_params=pltpu.CompilerParams(
            use_tc_tiling_on_sc=False, needs_layout_passes=False),
    )


def kernel(x, table, gamma, beta):
    b, l = x.shape
    chunk = 256
    tt = _make_transpose_call()(table.T)
    tail = table[VBLK * 128:].reshape(-1, 128)
    tt = lax.dynamic_update_slice(tt, tail, (VBLK * D, 0))
    tt2 = tt.reshape(2 * V, 32)
    xt = x.T.reshape(l, NW, b // (NW * chunk), chunk).transpose(1, 0, 2, 3)
    out = _make_gather_call(b, l, chunk)(xt, tt2, gamma, beta)
    return out.transpose(2, 0, 1)


# speedup vs baseline: 1.7287x; 1.2354x over previous
import jax
import jax.numpy as jnp
from jax import lax
from jax.experimental import pallas as pl
from jax.experimental.pallas import tpu as pltpu
from jax.experimental.pallas import tpu_sc as plsc

D = 64
LN_EPS = 1e-5
NC = 2
NS = 16
NW = NC * NS
TPAD = 273

_GATHER_DNUMS = lax.GatherDimensionNumbers(
    offset_dims=(), collapsed_slice_dims=(0,), start_index_map=(0,))


def _lane_sum(x):
    lane = lax.iota(jnp.int32, 16)
    for s in (1, 2, 4, 8):
        p = (lane ^ s).reshape(16, 1)
        x = x + lax.gather(x, p, _GATHER_DNUMS, (1,),
                           mode=lax.GatherScatterMode.PROMISE_IN_BOUNDS)
    return x


def _ln_row_t(gbuf, tbuf, r, g_regs, b_regs, kbase):
    v = [gbuf[r, pl.ds(16 * k, 16)] for k in range(4)]
    s = _lane_sum(v[0] + v[1] + v[2] + v[3])
    ss = _lane_sum(v[0] * v[0] + (v[1] * v[1] + (v[2] * v[2] + v[3] * v[3])))
    mean = s * (1.0 / 64.0)
    var = ss * (1.0 / 64.0) - mean * mean
    x = var + LN_EPS
    i = lax.bitcast_convert_type(x, jnp.int32)
    i = jnp.int32(0x5F3759DF) - lax.shift_right_logical(i, 1)
    y = lax.bitcast_convert_type(i, jnp.float32)
    xh = 0.5 * x
    y = y * (1.5 - xh * y * y)
    y = y * (1.5 - xh * y * y)
    nb = -mean * y
    rv = jnp.full((16,), r, jnp.int32)
    for k in range(4):
        o = (v[k] * y + nb) * g_regs[k] + b_regs[k]
        plsc.store_scatter(tbuf, [kbase[k], rv], o)


def _make_sc_call(batch, n_l, chunk):
    stripe = batch // NW
    halves = stripe // chunk
    n_chunks = n_l * halves
    n_cols = batch // chunk
    mesh = plsc.VectorSubcoreMesh(core_axis_name="c", subcore_axis_name="s")

    def body(x_hbm, tab_hbm, g_hbm, b_hbm, out_hbm,
             idx_all, g0, g1, t0, t1, g_v, b_v,
             sg0, sg1, so0, so1):
        wid = lax.axis_index("s") * NC + lax.axis_index("c")
        pltpu.sync_copy(x_hbm.at[wid], idx_all)
        pltpu.sync_copy(g_hbm, g_v)
        pltpu.sync_copy(b_hbm, b_v)
        g_regs = [g_v[pl.ds(16 * k, 16)] for k in range(4)]
        b_regs = [b_v[pl.ds(16 * k, 16)] for k in range(4)]
        lane = lax.iota(jnp.int32, 16)
        kbase = [lane + (16 * k) for k in range(4)]
        bufs = ((g0, t0, sg0, so0), (g1, t1, sg1, so1))

        def fire_gather(c, g, sg):
            pltpu.async_copy(
                tab_hbm.at[idx_all.at[c >> 1, c & (halves - 1)]], g, sg)

        def wait_gather(c, g, sg):
            pltpu.make_async_copy(
                tab_hbm.at[idx_all.at[c >> 1, c & (halves - 1)]], g, sg
            ).wait()

        def out_slice(c):
            col = wid * halves + (c & (halves - 1))
            return out_hbm.at[c >> 1, :, col, :]

        def fire_out(c, t, so):
            pltpu.async_copy(t.at[:, pl.ds(0, chunk)], out_slice(c), so)

        def wait_out(c, t, so):
            pltpu.make_async_copy(
                t.at[:, pl.ds(0, chunk)], out_slice(c), so).wait()

        def compute(g, t):
            @plsc.parallel_loop(0, chunk, unroll=8)
            def _row(r):
                _ln_row_t(g, t, r, g_regs, b_regs, kbase)

        def step(c, p, first):
            g_m, t_m, sg_m, so_m = bufs[p]
            g_o, t_o, sg_o, so_o = bufs[1 - p]
            if not first:
                wait_out(c - 1, t_o, so_o)

            @pl.when(c + 1 < n_chunks)
            def _():
                fire_gather(c + 1, g_o, sg_o)

            wait_gather(c, g_m, sg_m)
            compute(g_m, t_m)
            fire_out(c, t_m, so_m)

        fire_gather(0, g0, sg0)
        step(0, 0, first=True)

        @pl.loop(1, n_chunks)
        def _chunk(c):
            @pl.when((c & 1) == 1)
            def _():
                step(c, 1, first=False)

            @pl.when((c & 1) == 0)
            def _():
                step(c, 0, first=False)

        last = (n_chunks - 1) & 1
        wait_out(n_chunks - 1, bufs[last][1], bufs[last][3])

    return pl.kernel(
        body,
        out_type=jax.ShapeDtypeStruct((n_l, D, n_cols, chunk), jnp.float32),
        mesh=mesh,
        scratch_types=[
            pltpu.VMEM((n_l, halves, chunk), jnp.int32),
            pltpu.VMEM((chunk, D), jnp.float32),
            pltpu.VMEM((chunk, D), jnp.float32),
            pltpu.VMEM((D, TPAD), jnp.float32),
            pltpu.VMEM((D, TPAD), jnp.float32),
            pltpu.VMEM((D,), jnp.float32),
            pltpu.VMEM((D,), jnp.float32),
            pltpu.SemaphoreType.DMA,
            pltpu.SemaphoreType.DMA,
            pltpu.SemaphoreType.DMA,
            pltpu.SemaphoreType.DMA,
        ],
        compiler_params=pltpu.CompilerParams(
            use_tc_tiling_on_sc=False, needs_layout_passes=False),
    )


def kernel(x, table, gamma, beta):
    b, l = x.shape
    chunk = 256
    xt = x.T.reshape(l, NW, b // (NW * chunk), chunk).transpose(1, 0, 2, 3)
    out = _make_sc_call(b, l, chunk)(xt, table, gamma, beta)
    return out.reshape(l, D, b).transpose(2, 0, 1)


# speedup vs baseline: 1.9039x; 1.1014x over previous
import jax
import jax.numpy as jnp
from jax import lax
from jax.experimental import pallas as pl
from jax.experimental.pallas import tpu as pltpu
from jax.experimental.pallas import tpu_sc as plsc

D = 64
LN_EPS = 1e-5
NC = 2
NS = 16
NW = NC * NS
TPAD = 273

_GATHER_DNUMS = lax.GatherDimensionNumbers(
    offset_dims=(), collapsed_slice_dims=(0,), start_index_map=(0,))


def _lane_sum(x):
    lane = lax.iota(jnp.int32, 16)
    for s in (1, 2, 4, 8):
        p = (lane ^ s).reshape(16, 1)
        x = x + lax.gather(x, p, _GATHER_DNUMS, (1,),
                           mode=lax.GatherScatterMode.PROMISE_IN_BOUNDS)
    return x


def _ln_row_t(gbuf, tbuf, r, g_regs, b_regs, kbase):
    v = [gbuf[r, pl.ds(16 * k, 16)] for k in range(4)]
    s = _lane_sum(v[0] + v[1] + v[2] + v[3])
    ss = _lane_sum(v[0] * v[0] + (v[1] * v[1] + (v[2] * v[2] + v[3] * v[3])))
    mean = s * (1.0 / 64.0)
    var = ss * (1.0 / 64.0) - mean * mean
    x = var + LN_EPS
    i = lax.bitcast_convert_type(x, jnp.int32)
    i = jnp.int32(0x5F3759DF) - lax.shift_right_logical(i, 1)
    y = lax.bitcast_convert_type(i, jnp.float32)
    xh = 0.5 * x
    y = y * (1.5 - xh * y * y)
    y = y * (1.5 - xh * y * y)
    nb = -mean * y
    rv = jnp.full((16,), r, jnp.int32)
    for k in range(4):
        o = (v[k] * y + nb) * g_regs[k] + b_regs[k]
        plsc.store_scatter(tbuf, [kbase[k], rv], o)


def _make_sc_call(batch, n_l, chunk):
    stripe = batch // NW
    halves = stripe // chunk
    n_chunks = n_l * halves
    n_cols = batch // chunk
    mesh = plsc.VectorSubcoreMesh(core_axis_name="c", subcore_axis_name="s")

    def body(x_hbm, tab_hbm, g_hbm, b_hbm, out_hbm,
             idx_all, g0, g1, t0, t1, g_v, b_v,
             sg0, sg1, so0, so1):
        wid = lax.axis_index("s") * NC + lax.axis_index("c")
        pltpu.sync_copy(x_hbm.at[wid], idx_all)
        pltpu.sync_copy(g_hbm, g_v)
        pltpu.sync_copy(b_hbm, b_v)
        g_regs = [g_v[pl.ds(16 * k, 16)] for k in range(4)]
        b_regs = [b_v[pl.ds(16 * k, 16)] for k in range(4)]
        lane = lax.iota(jnp.int32, 16)
        kbase = [lane + (16 * k) for k in range(4)]
        bufs = ((g0, t0, sg0, so0), (g1, t1, sg1, so1))

        def fire_gather(c, g, sg):
            pltpu.async_copy(
                tab_hbm.at[idx_all.at[c >> 1, c & (halves - 1)]], g, sg)

        def wait_gather(c, g, sg):
            pltpu.make_async_copy(
                tab_hbm.at[idx_all.at[c >> 1, c & (halves - 1)]], g, sg
            ).wait()

        def out_slice(c):
            col = wid * halves + (c & (halves - 1))
            return out_hbm.at[c >> 1, :, col, :]

        def fire_out(c, t, so):
            pltpu.async_copy(t.at[:, pl.ds(0, chunk)], out_slice(c), so)

        def wait_out(c, t, so):
            pltpu.make_async_copy(
                t.at[:, pl.ds(0, chunk)], out_slice(c), so).wait()

        def compute(g, t):
            @plsc.parallel_loop(0, chunk, unroll=4)
            def _row(r):
                _ln_row_t(g, t, r, g_regs, b_regs, kbase)

        def step(c, p, first):
            g_m, t_m, sg_m, so_m = bufs[p]
            g_o, t_o, sg_o, so_o = bufs[1 - p]
            if not first:
                wait_out(c - 1, t_o, so_o)

            @pl.when(c + 1 < n_chunks)
            def _():
                fire_gather(c + 1, g_o, sg_o)

            wait_gather(c, g_m, sg_m)
            compute(g_m, t_m)
            fire_out(c, t_m, so_m)

        fire_gather(0, g0, sg0)
        step(0, 0, first=True)

        @pl.loop(1, n_chunks)
        def _chunk(c):
            @pl.when((c & 1) == 1)
            def _():
                step(c, 1, first=False)

            @pl.when((c & 1) == 0)
            def _():
                step(c, 0, first=False)

        last = (n_chunks - 1) & 1
        wait_out(n_chunks - 1, bufs[last][1], bufs[last][3])

    return pl.kernel(
        body,
        out_type=jax.ShapeDtypeStruct((n_l, D, n_cols, chunk), jnp.float32),
        mesh=mesh,
        scratch_types=[
            pltpu.VMEM((n_l, halves, chunk), jnp.int32),
            pltpu.VMEM((chunk, D), jnp.float32),
            pltpu.VMEM((chunk, D), jnp.float32),
            pltpu.VMEM((D, TPAD), jnp.float32),
            pltpu.VMEM((D, TPAD), jnp.float32),
            pltpu.VMEM((D,), jnp.float32),
            pltpu.VMEM((D,), jnp.float32),
            pltpu.SemaphoreType.DMA,
            pltpu.SemaphoreType.DMA,
            pltpu.SemaphoreType.DMA,
            pltpu.SemaphoreType.DMA,
        ],
        compiler_params=pltpu.CompilerParams(
            use_tc_tiling_on_sc=False, needs_layout_passes=False),
    )


def kernel(x, table, gamma, beta):
    b, l = x.shape
    chunk = 256
    xt = x.T.reshape(l, NW, b // (NW * chunk), chunk).transpose(1, 0, 2, 3)
    out = _make_sc_call(b, l, chunk)(xt, table, gamma, beta)
    return out.reshape(l, D, b).transpose(2, 0, 1)
